# baseline (speedup 1.0000x reference)
import jax
import jax.numpy as jnp
from jax import lax
from jax.experimental import pallas as pl
from jax.experimental.pallas import tpu as pltpu
from jax.experimental.pallas import tpu_sc as plsc

NE = 64
NTOK = 2048
D = 768
BT = 128
NB = NTOK // BT
S = NB + NE
NC = 2
NS = 16
NW = NC * NS
TPW = NTOK // NW
L = 16
NTP = (NB + 1) * BT
BIG = 2**30


def _gate_body(x_ref, gw_ref, gb_ref, pos_ref, prob_ref, meta_ref):
    x = x_ref[...]
    scores = lax.dot_general(x, gw_ref[...], (((1,), (1,)), ((), ())),
                             preferred_element_type=jnp.float32)
    scores = scores + gb_ref[...]
    m = jnp.max(scores, axis=1, keepdims=True)
    s = jnp.sum(jnp.exp(scores - m), axis=1, keepdims=True)
    prob_ref[...] = 1.0 / s
    lane = lax.broadcasted_iota(jnp.int32, scores.shape, 1)
    eid = jnp.min(jnp.where(scores == m, lane, NE), axis=1, keepdims=True)
    oh = (lane == eid).astype(jnp.float32)
    counts = jnp.sum(oh, axis=0, keepdims=True)
    ei = lax.broadcasted_iota(jnp.int32, (NE, NE), 0)
    ej = lax.broadcasted_iota(jnp.int32, (NE, NE), 1)
    sum_mask = ei < ej

    def excl_prefix(v):
        vb = jnp.broadcast_to(v.reshape(NE, 1), (NE, NE))
        return jnp.sum(jnp.where(sum_mask, vb, 0.0), axis=0, keepdims=True)

    starts = excl_prefix(counts)
    ri = lax.broadcasted_iota(jnp.int32, (BT, BT), 0)
    rj = lax.broadcasted_iota(jnp.int32, (BT, BT), 1)
    tl = (rj < ri).astype(jnp.float32)
    run = jnp.zeros((1, NE), jnp.float32)
    for b in range(NB):
        blk = oh[b * BT:(b + 1) * BT]
        rank = lax.dot_general(tl, blk, (((1,), (0,)), ((), ())),
                               preferred_element_type=jnp.float32) + run
        pos = jnp.sum(blk * (starts + rank), axis=1, keepdims=True)
        pos_ref[b * BT:(b + 1) * BT, :] = pos.astype(jnp.int32)
        run = run + jnp.sum(blk, axis=0, keepdims=True)
    ends = starts + counts
    lob = jnp.floor(starts * (1.0 / BT))
    hib = jnp.floor((ends - 1.0) * (1.0 / BT))
    nonempty = counts > 0.0
    p = jnp.where(nonempty, hib - lob + 1.0, 0.0)
    q = excl_prefix(p)
    r = jnp.where(nonempty, q, 1e6)
    sv = lax.broadcasted_iota(jnp.int32, (S, 1), 0).astype(jnp.float32)
    es = jnp.sum((jnp.broadcast_to(r, (S, NE)) <= sv).astype(jnp.float32),
                 axis=1, keepdims=True) - 1.0
    se = lax.broadcasted_iota(jnp.int32, (S, NE), 1).astype(jnp.float32)
    ohb = se == es

    def pick(v):
        return jnp.sum(jnp.where(ohb, jnp.broadcast_to(v, (S, NE)), 0.0),
                       axis=1, keepdims=True)

    qs, ps_, lobs, sts, ens = (pick(q), pick(p), pick(lob), pick(starts),
                               pick(ends))
    j = sv - qs
    valid = j < ps_
    bs = jnp.where(valid, lobs + j, float(NB - 1))
    los = jnp.where(valid, jnp.maximum(sts, bs * BT), 0.0)
    his = jnp.where(valid, jnp.minimum(ens, (bs + 1.0) * BT), 0.0)
    meta = jnp.concatenate([bs, es, los, his], axis=1)
    meta_ref[...] = meta.astype(jnp.int32)


def _gate(x, gate_w, gate_b):
    return pl.pallas_call(
        _gate_body,
        out_shape=[
            jax.ShapeDtypeStruct((NTOK, 1), jnp.int32),
            jax.ShapeDtypeStruct((NTOK, 1), jnp.float32),
            jax.ShapeDtypeStruct((S, 4), jnp.int32),
        ],
    )(x, gate_w, gate_b.reshape(1, NE))


def _sc_mesh():
    return plsc.VectorSubcoreMesh(core_axis_name="c", subcore_axis_name="s",
                                  num_cores=NC, num_subcores=NS)


def _dispatch_body(x_hbm, pos_hbm, prob_hbm, xs_hbm, ps_hbm,
                   pos_v, pstage, xrows_v, sem_a, sem_b):
    wid = lax.axis_index("s") * NC + lax.axis_index("c")
    base = wid * TPW
    c1 = pltpu.async_copy(x_hbm.at[pl.ds(base, TPW)], xrows_v, sem_a)
    c2 = pltpu.async_copy(pos_hbm.at[pl.ds(base, TPW)], pos_v, sem_b)
    c3 = pltpu.async_copy(prob_hbm.at[pl.ds(base, TPW)], pstage, sem_b)
    c1.wait()
    c2.wait()
    c3.wait()
    c4 = pltpu.async_copy(xrows_v, xs_hbm.at[pos_v], sem_a)
    c5 = pltpu.async_copy(pstage, ps_hbm.at[pos_v], sem_b)
    c4.wait()
    c5.wait()


def _dispatch(x, pos, prob):
    f = pl.kernel(
        _dispatch_body,
        out_type=(
            jax.ShapeDtypeStruct((NTP, D), jnp.float32),
            jax.ShapeDtypeStruct((NTP,), jnp.float32),
        ),
        mesh=_sc_mesh(),
        scratch_types=[
            pltpu.VMEM((TPW,), jnp.int32),
            pltpu.VMEM((TPW,), jnp.float32),
            pltpu.VMEM((TPW, D), jnp.float32),
            pltpu.SemaphoreType.DMA,
            pltpu.SemaphoreType.DMA,
        ],
    )
    return f(x, pos, prob)


def _mm_body(meta_ref, xs_ref, w_ref, b_ref, ps_ref, y_ref):
    s = pl.program_id(0)
    b = meta_ref[s, 0]
    lo = meta_ref[s, 2]
    hi = meta_ref[s, 3]
    rows = b * BT + lax.broadcasted_iota(jnp.int32, (BT, 1), 0)
    mask = (rows >= lo) & (rows < hi)
    y = lax.dot_general(xs_ref[...], w_ref[0], (((1,), (1,)), ((), ())),
                        preferred_element_type=jnp.float32)
    y = (y + b_ref[0]) * ps_ref[...]
    y_ref[...] = jnp.where(mask, y, y_ref[...])


def _grouped_mm(xs, ps, expert_w, expert_b, meta):
    grid_spec = pltpu.PrefetchScalarGridSpec(
        num_scalar_prefetch=1,
        grid=(S,),
        in_specs=[
            pl.BlockSpec((BT, D), lambda s, meta: (meta[s, 0], 0)),
            pl.BlockSpec((1, D, D), lambda s, meta: (meta[s, 1], 0, 0)),
            pl.BlockSpec((1, 1, D), lambda s, meta: (meta[s, 1], 0, 0)),
            pl.BlockSpec((BT, 1), lambda s, meta: (meta[s, 0], 0)),
        ],
        out_specs=pl.BlockSpec((BT, D), lambda s, meta: (meta[s, 0], 0)),
    )
    return pl.pallas_call(
        _mm_body,
        grid_spec=grid_spec,
        out_shape=jax.ShapeDtypeStruct((NTP, D), jnp.float32),
    )(meta, xs, expert_w, expert_b.reshape(NE, 1, D), ps.reshape(NTP, 1))


def _combine_body(y_hbm, pos_hbm, out_hbm, pos_v, rows_v, sem_a):
    wid = lax.axis_index("s") * NC + lax.axis_index("c")
    base = wid * TPW
    pltpu.sync_copy(pos_hbm.at[pl.ds(base, TPW)], pos_v)
    pltpu.async_copy(y_hbm.at[pos_v], rows_v, sem_a).wait()
    pltpu.sync_copy(rows_v, out_hbm.at[pl.ds(base, TPW)])


def _combine(y, pos):
    f = pl.kernel(
        _combine_body,
        out_type=jax.ShapeDtypeStruct((NTOK, D), jnp.float32),
        mesh=_sc_mesh(),
        scratch_types=[
            pltpu.VMEM((TPW,), jnp.int32),
            pltpu.VMEM((TPW, D), jnp.float32),
            pltpu.SemaphoreType.DMA,
        ],
    )
    return f(y, pos)


def kernel(x, gate_w, gate_b, expert_w, expert_b):
    pos2, prob2, meta = _gate(x, gate_w, gate_b)
    pos = pos2[:, 0]
    xs, ps = _dispatch(x, pos, prob2[:, 0])
    y = _grouped_mm(xs, ps, expert_w, expert_b, meta)
    return _combine(y, pos)


# speedup vs baseline: 1.0965x; 1.0965x over previous
import jax
import jax.numpy as jnp
from jax import lax
from jax.experimental import pallas as pl
from jax.experimental.pallas import tpu as pltpu
from jax.experimental.pallas import tpu_sc as plsc

NE = 64
NTOK = 2048
D = 768
BT = 128
NB = NTOK // BT
S = NB + NE
NC = 2
NS = 16
NW = NC * NS
TPW = NTOK // NW
L = 16
NTP = (NB + 1) * BT
BIG = 2**30


def _gate_body(x_ref, gw_ref, gb_ref, pos_ref, prob_ref, meta_ref, inv_ref):
    x = x_ref[...]
    scores = lax.dot_general(x, gw_ref[...], (((1,), (1,)), ((), ())),
                             preferred_element_type=jnp.float32)
    scores = scores + gb_ref[...]
    m = jnp.max(scores, axis=1, keepdims=True)
    s = jnp.sum(jnp.exp(scores - m), axis=1, keepdims=True)
    prob_ref[...] = 1.0 / s
    lane = lax.broadcasted_iota(jnp.int32, scores.shape, 1)
    eid = jnp.min(jnp.where(scores == m, lane, NE), axis=1, keepdims=True)
    oh = (lane == eid).astype(jnp.float32)
    counts = jnp.sum(oh, axis=0, keepdims=True)
    ei = lax.broadcasted_iota(jnp.int32, (NE, NE), 0)
    ej = lax.broadcasted_iota(jnp.int32, (NE, NE), 1)
    sum_mask = ei < ej

    def excl_prefix(v):
        vb = jnp.broadcast_to(v.reshape(NE, 1), (NE, NE))
        return jnp.sum(jnp.where(sum_mask, vb, 0.0), axis=0, keepdims=True)

    starts = excl_prefix(counts)
    ri = lax.broadcasted_iota(jnp.int32, (BT, BT), 0)
    rj = lax.broadcasted_iota(jnp.int32, (BT, BT), 1)
    tl = (rj < ri).astype(jnp.float32)
    run = jnp.zeros((1, NE), jnp.float32)
    for b in range(NB):
        blk = oh[b * BT:(b + 1) * BT]
        rank = lax.dot_general(tl, blk, (((1,), (0,)), ((), ())),
                               preferred_element_type=jnp.float32) + run
        pos = jnp.sum(blk * (starts + rank), axis=1, keepdims=True)
        pos_ref[b * BT:(b + 1) * BT, :] = pos.astype(jnp.int32)
        run = run + jnp.sum(blk, axis=0, keepdims=True)
    ends = starts + counts
    lob = jnp.floor(starts * (1.0 / BT))
    hib = jnp.floor((ends - 1.0) * (1.0 / BT))
    nonempty = counts > 0.0
    p = jnp.where(nonempty, hib - lob + 1.0, 0.0)
    q = excl_prefix(p)
    r = jnp.where(nonempty, q, 1e6)
    sv = lax.broadcasted_iota(jnp.int32, (S, 1), 0).astype(jnp.float32)
    es = jnp.sum((jnp.broadcast_to(r, (S, NE)) <= sv).astype(jnp.float32),
                 axis=1, keepdims=True) - 1.0
    se = lax.broadcasted_iota(jnp.int32, (S, NE), 1).astype(jnp.float32)
    ohb = se == es

    def pick(v):
        return jnp.sum(jnp.where(ohb, jnp.broadcast_to(v, (S, NE)), 0.0),
                       axis=1, keepdims=True)

    qs, ps_, lobs, sts, ens = (pick(q), pick(p), pick(lob), pick(starts),
                               pick(ends))
    j = sv - qs
    valid = j < ps_
    bs = jnp.where(valid, lobs + j, float(NB - 1))
    los = jnp.where(valid, jnp.maximum(sts, bs * BT), 0.0)
    his = jnp.where(valid, jnp.minimum(ens, (bs + 1.0) * BT), 0.0)
    meta = jnp.concatenate([bs, es, los, his], axis=1)
    meta_ref[...] = meta.astype(jnp.int32)
    pos_all = pos_ref[...].astype(jnp.float32)
    tvec = lax.broadcasted_iota(jnp.int32, (NTOK, 1), 0)
    tsplit = jnp.concatenate(
        [(tvec % BT).astype(jnp.float32), (tvec // BT).astype(jnp.float32)],
        axis=1)
    srow = lax.broadcasted_iota(jnp.int32, (1, BT), 1).astype(jnp.float32)
    for sb in range(NB):
        cmp = (pos_all == (srow + sb * BT)).astype(jnp.float32)
        iv = lax.dot_general(cmp, tsplit, (((0,), (0,)), ((), ())),
                             preferred_element_type=jnp.float32)
        inv = iv[:, 0:1] + iv[:, 1:2] * BT
        inv_ref[sb * BT:(sb + 1) * BT, :] = inv.astype(jnp.int32)


def _gate(x, gate_w, gate_b):
    return pl.pallas_call(
        _gate_body,
        out_shape=[
            jax.ShapeDtypeStruct((NTOK, 1), jnp.int32),
            jax.ShapeDtypeStruct((NTOK, 1), jnp.float32),
            jax.ShapeDtypeStruct((S, 4), jnp.int32),
            jax.ShapeDtypeStruct((NTOK, 1), jnp.int32),
        ],
    )(x, gate_w, gate_b.reshape(1, NE))


def _sc_mesh():
    return plsc.VectorSubcoreMesh(core_axis_name="c", subcore_axis_name="s",
                                  num_cores=NC, num_subcores=NS)


def _dispatch_body(x_hbm, inv_hbm, prob_hbm, xs_hbm, ps_hbm,
                   inv_v, pstage, xrows_v, sem_a, sem_b):
    wid = lax.axis_index("s") * NC + lax.axis_index("c")
    base = wid * TPW
    pltpu.sync_copy(inv_hbm.at[pl.ds(base, TPW)], inv_v)
    c1 = pltpu.async_copy(x_hbm.at[inv_v], xrows_v, sem_a)
    c2 = pltpu.async_copy(prob_hbm.at[inv_v], pstage, sem_b)
    c1.wait()
    c2.wait()
    c3 = pltpu.async_copy(xrows_v, xs_hbm.at[pl.ds(base, TPW)], sem_a)
    c4 = pltpu.async_copy(pstage, ps_hbm.at[pl.ds(base, TPW)], sem_b)
    c3.wait()
    c4.wait()


def _dispatch(x, inv, prob):
    f = pl.kernel(
        _dispatch_body,
        out_type=(
            jax.ShapeDtypeStruct((NTP, D), jnp.float32),
            jax.ShapeDtypeStruct((NTP,), jnp.float32),
        ),
        mesh=_sc_mesh(),
        scratch_types=[
            pltpu.VMEM((TPW,), jnp.int32),
            pltpu.VMEM((TPW,), jnp.float32),
            pltpu.VMEM((TPW, D), jnp.float32),
            pltpu.SemaphoreType.DMA,
            pltpu.SemaphoreType.DMA,
        ],
    )
    return f(x, inv, prob)


def _mm_body(meta_ref, xs_ref, w_ref, b_ref, ps_ref, y_ref):
    s = pl.program_id(0)
    b = meta_ref[s, 0]
    lo = meta_ref[s, 2]
    hi = meta_ref[s, 3]
    rows = b * BT + lax.broadcasted_iota(jnp.int32, (BT, 1), 0)
    mask = (rows >= lo) & (rows < hi)
    y = lax.dot_general(xs_ref[...], w_ref[0], (((1,), (1,)), ((), ())),
                        preferred_element_type=jnp.float32)
    y = (y + b_ref[0]) * ps_ref[...]
    y_ref[...] = jnp.where(mask, y, y_ref[...])


def _grouped_mm(xs, ps, expert_w, expert_b, meta):
    grid_spec = pltpu.PrefetchScalarGridSpec(
        num_scalar_prefetch=1,
        grid=(S,),
        in_specs=[
            pl.BlockSpec((BT, D), lambda s, meta: (meta[s, 0], 0)),
            pl.BlockSpec((1, D, D), lambda s, meta: (meta[s, 1], 0, 0)),
            pl.BlockSpec((1, 1, D), lambda s, meta: (meta[s, 1], 0, 0)),
            pl.BlockSpec((BT, 1), lambda s, meta: (meta[s, 0], 0)),
        ],
        out_specs=pl.BlockSpec((BT, D), lambda s, meta: (meta[s, 0], 0)),
    )
    return pl.pallas_call(
        _mm_body,
        grid_spec=grid_spec,
        out_shape=jax.ShapeDtypeStruct((NTP, D), jnp.float32),
    )(meta, xs, expert_w, expert_b.reshape(NE, 1, D), ps.reshape(NTP, 1))


def _combine_body(y_hbm, pos_hbm, out_hbm, pos_v, rows_v, sem_a):
    wid = lax.axis_index("s") * NC + lax.axis_index("c")
    base = wid * TPW
    pltpu.sync_copy(pos_hbm.at[pl.ds(base, TPW)], pos_v)
    pltpu.async_copy(y_hbm.at[pos_v], rows_v, sem_a).wait()
    pltpu.sync_copy(rows_v, out_hbm.at[pl.ds(base, TPW)])


def _combine(y, pos):
    f = pl.kernel(
        _combine_body,
        out_type=jax.ShapeDtypeStruct((NTOK, D), jnp.float32),
        mesh=_sc_mesh(),
        scratch_types=[
            pltpu.VMEM((TPW,), jnp.int32),
            pltpu.VMEM((TPW, D), jnp.float32),
            pltpu.SemaphoreType.DMA,
        ],
    )
    return f(y, pos)


def kernel(x, gate_w, gate_b, expert_w, expert_b):
    pos2, prob2, meta, inv2 = _gate(x, gate_w, gate_b)
    xs, ps = _dispatch(x, inv2[:, 0], prob2[:, 0])
    y = _grouped_mm(xs, ps, expert_w, expert_b, meta)
    return _combine(y, pos2[:, 0])
